# Initial kernel scaffold; baseline (speedup 1.0000x reference)
import jax
import jax.numpy as jnp
from jax.experimental import pallas as pl


def kernel(adjm, node_feats, trip_od, W1s, W2s, W1d, W2d, Win, bin_, Wout, bout):
    raise NotImplementedError("write your pallas kernel here")


# speedup vs baseline: 3.7513x; 3.7513x over previous
import functools

import jax
import jax.numpy as jnp
from jax import lax
from jax.experimental import pallas as pl
from jax.experimental.pallas import tpu as pltpu
from jax.experimental.pallas import tpu_sc as plsc

N = 10000
NT = 320000
HID = 32
OUT_DIM = 24

NS = 16
NC = 2
CH = 640
NPAD = NS * CH
TI = NT // NS
BM = 400


def _embed_body(adj_ref, nf_ref, w1s_ref, w2s_ref, w1d_ref, w2d_ref,
                win_ref, bin_ref, wout_ref, bout_ref, si_ref, so_ref):
    h = jnp.dot(adj_ref[...], nf_ref[...], preferred_element_type=jnp.float32)
    vs = jnp.dot(w2s_ref[...], wout_ref[...], preferred_element_type=jnp.float32)
    vd = jnp.dot(w2d_ref[...], win_ref[...], preferred_element_type=jnp.float32)
    a_s = jnp.tanh(h * w1s_ref[...])
    a_d = jnp.tanh(h * w1d_ref[...])
    so_ref[...] = jnp.dot(a_s, vs, preferred_element_type=jnp.float32) + bout_ref[0, 0]
    si_ref[...] = jnp.dot(a_d, vd, preferred_element_type=jnp.float32) + bin_ref[0, 0]


def _embed_call(adjm, nf, W1s, W2s, W1d, W2d, Win, bin2, Wout, bout2):
    const = lambda i: (0, 0)
    return pl.pallas_call(
        _embed_body,
        grid=(N // BM,),
        in_specs=[
            pl.BlockSpec((BM, N), lambda i: (i, 0)),
            pl.BlockSpec((N, 1), const),
            pl.BlockSpec((1, HID), const),
            pl.BlockSpec((HID, OUT_DIM), const),
            pl.BlockSpec((1, HID), const),
            pl.BlockSpec((HID, OUT_DIM), const),
            pl.BlockSpec((OUT_DIM, 1), const),
            pl.BlockSpec((1, 1), const),
            pl.BlockSpec((OUT_DIM, 1), const),
            pl.BlockSpec((1, 1), const),
        ],
        out_specs=[pl.BlockSpec((BM, 1), lambda i: (i, 0)),
                   pl.BlockSpec((BM, 1), lambda i: (i, 0))],
        out_shape=[jax.ShapeDtypeStruct((N, 1), jnp.float32),
                   jax.ShapeDtypeStruct((N, 1), jnp.float32)],
    )(adjm, nf, W1s, W2s, W1d, W2d, Win, bin2, Wout, bout2)


def _sc_body(idx_hbm, scal_hbm, out_hbm, idx_v, pres_v, slice_all,
             pres_chunk, scal_v, compact_v, pos2d, s0_buf, fill_v,
             tot_buf, totals_local, presence_sp, totals_sp):
    c = lax.axis_index("c")
    t = lax.axis_index("s")
    iota = lax.iota(jnp.int32, 16)
    zero16 = jnp.zeros((16,), jnp.int32)
    one16 = jnp.ones((16,), jnp.int32)

    pltpu.sync_copy(idx_hbm.at[pl.ds(c * NT + t * TI, TI)], idx_v)
    pltpu.sync_copy(scal_hbm.at[pl.ds(c * NPAD + t * CH, CH)], scal_v)
    pltpu.sync_copy(scal_hbm.at[pl.ds(c * NPAD, 16)], s0_buf)

    def _z(i, carry):
        pres_v[pl.ds(i * 16, 16)] = zero16
        return carry
    lax.fori_loop(0, NPAD // 16, _z, 0)

    def _scat(i, carry):
        iv = idx_v[pl.ds(i * 16, 16)]
        plsc.store_scatter(pres_v, [iv], one16)
        return carry
    lax.fori_loop(0, TI // 16, _scat, 0)

    s0 = jnp.sum(jnp.where(iota == 0, s0_buf[...], 0.0))
    fvec = jnp.broadcast_to(s0, (16,))
    def _f(i, carry):
        fill_v[pl.ds(i * 16, 16)] = fvec
        return carry
    lax.fori_loop(0, CH // 16, _f, 0)
    pltpu.sync_copy(fill_v, out_hbm.at[pl.ds(c * NPAD + t * CH, CH)])

    pltpu.sync_copy(pres_v, presence_sp.at[pl.ds(t * NPAD, NPAD)])
    plsc.subcore_barrier()

    for u in range(NS):
        pltpu.sync_copy(presence_sp.at[pl.ds(u * NPAD + t * CH, CH)],
                        slice_all.at[pl.ds(u * CH, CH)])

    g0 = t * CH
    def _merge(j, tot):
        acc = zero16
        for u in range(NS):
            acc = acc | slice_all[pl.ds(u * CH + j * 16, 16)]
        m = (acc > 0) & (g0 + j * 16 + iota < N)
        mi = m.astype(jnp.int32)
        pres_chunk[pl.ds(j * 16, 16)] = mi
        return tot + jnp.sum(mi)
    total_t = lax.fori_loop(0, CH // 16, _merge, jnp.int32(0))

    tot_buf[...] = jnp.broadcast_to(total_t, (16,))
    pltpu.sync_copy(tot_buf, totals_sp.at[pl.ds(t * 16, 16)])
    plsc.subcore_barrier()

    pltpu.sync_copy(totals_sp, totals_local)
    totals_v = plsc.load_gather(totals_local, [iota * 16])
    off0 = jnp.sum(jnp.where(iota < t, totals_v, 0))
    base = c * NPAD + off0
    dump = c * NPAD + N + t

    for k in range(CH // 16):
        lk = k * 16 + iota
        p = jnp.where(lk < total_t, base + lk, dump)
        pos2d[k // 8, pl.ds((k % 8) * 16, 16)] = p

    def _comp(j, off):
        m = pres_chunk[pl.ds(j * 16, 16)] > 0
        sc = scal_v[pl.ds(j * 16, 16)]
        mi = m.astype(jnp.int32)
        rank = plsc.cumsum(mi)
        posl = jnp.where(m, off + rank - 1, 0)
        plsc.store_scatter(compact_v, [posl], sc, mask=m)
        return off + jnp.sum(mi)
    lax.fori_loop(0, CH // 16, _comp, jnp.int32(0))

    for r in range(CH // 128):
        pltpu.sync_copy(compact_v.at[pl.ds(r * 128, 128)],
                        out_hbm.at[pos2d.at[r]])


@functools.lru_cache(maxsize=1)
def _sc_unique_gather():
    mesh = plsc.VectorSubcoreMesh(core_axis_name="c", subcore_axis_name="s",
                                  num_cores=NC, num_subcores=NS)
    return pl.kernel(
        _sc_body,
        out_type=jax.ShapeDtypeStruct((2 * NPAD,), jnp.float32),
        mesh=mesh,
        compiler_params=pltpu.CompilerParams(needs_layout_passes=False),
        scratch_types=[
            pltpu.VMEM((TI,), jnp.int32),
            pltpu.VMEM((NPAD,), jnp.int32),
            pltpu.VMEM((NS * CH,), jnp.int32),
            pltpu.VMEM((CH,), jnp.int32),
            pltpu.VMEM((CH,), jnp.float32),
            pltpu.VMEM((CH,), jnp.float32),
            pltpu.VMEM((CH // 128, 128), jnp.int32),
            pltpu.VMEM((16,), jnp.float32),
            pltpu.VMEM((CH,), jnp.float32),
            pltpu.VMEM((16,), jnp.int32),
            pltpu.VMEM((NS * 16,), jnp.int32),
            pltpu.VMEM_SHARED((NS * NPAD,), jnp.int32),
            pltpu.VMEM_SHARED((NS * 16,), jnp.int32),
        ],
    )


def kernel(adjm, node_feats, trip_od, W1s, W2s, W1d, W2d, Win, bin_, Wout, bout):
    scal_in, scal_out = _embed_call(
        adjm, node_feats, W1s, W2s, W1d, W2d,
        Win, bin_.reshape(1, 1), Wout, bout.reshape(1, 1))
    pad = NPAD - N
    scal2 = jnp.concatenate([
        jnp.pad(scal_in[:, 0], (0, pad)),
        jnp.pad(scal_out[:, 0], (0, pad)),
    ])
    idx2 = jnp.concatenate([trip_od[:, 1], trip_od[:, 0]]).astype(jnp.int32)
    out_flat = _sc_unique_gather()(idx2, scal2)
    return jnp.stack([out_flat[:N], out_flat[NPAD:NPAD + N]], axis=1)


# speedup vs baseline: 4.5145x; 1.2035x over previous
import functools

import jax
import jax.numpy as jnp
from jax import lax
from jax.experimental import pallas as pl
from jax.experimental.pallas import tpu as pltpu
from jax.experimental.pallas import tpu_sc as plsc

N = 10000
NT = 320000
HID = 32
OUT_DIM = 24

NS = 16
NC = 2
CH = 640
NPAD = NS * CH
TI = NT // NS
BM = 400


def _embed_body(adj_ref, nf_ref, w1s_ref, w2s_ref, w1d_ref, w2d_ref,
                win_ref, bin_ref, wout_ref, bout_ref, si_ref, so_ref):
    h = jnp.dot(adj_ref[...], nf_ref[...], preferred_element_type=jnp.float32)
    vs = jnp.dot(w2s_ref[...], wout_ref[...], preferred_element_type=jnp.float32)
    vd = jnp.dot(w2d_ref[...], win_ref[...], preferred_element_type=jnp.float32)
    a_s = jnp.tanh(h * w1s_ref[...])
    a_d = jnp.tanh(h * w1d_ref[...])
    so_ref[...] = jnp.dot(a_s, vs, preferred_element_type=jnp.float32) + bout_ref[0, 0]
    si_ref[...] = jnp.dot(a_d, vd, preferred_element_type=jnp.float32) + bin_ref[0, 0]


def _embed_call(adjm, nf, W1s, W2s, W1d, W2d, Win, bin2, Wout, bout2):
    const = lambda i: (0, 0)
    return pl.pallas_call(
        _embed_body,
        grid=(N // BM,),
        in_specs=[
            pl.BlockSpec((BM, N), lambda i: (i, 0)),
            pl.BlockSpec((N, 1), const),
            pl.BlockSpec((1, HID), const),
            pl.BlockSpec((HID, OUT_DIM), const),
            pl.BlockSpec((1, HID), const),
            pl.BlockSpec((HID, OUT_DIM), const),
            pl.BlockSpec((OUT_DIM, 1), const),
            pl.BlockSpec((1, 1), const),
            pl.BlockSpec((OUT_DIM, 1), const),
            pl.BlockSpec((1, 1), const),
        ],
        out_specs=[pl.BlockSpec((BM, 1), lambda i: (i, 0)),
                   pl.BlockSpec((BM, 1), lambda i: (i, 0))],
        out_shape=[jax.ShapeDtypeStruct((N, 1), jnp.float32),
                   jax.ShapeDtypeStruct((N, 1), jnp.float32)],
    )(adjm, nf, W1s, W2s, W1d, W2d, Win, bin2, Wout, bout2)


def _unique_body(idx_hbm, ids_hbm, idx_v, pres_v, slice_all, pres_chunk,
                 compact_v, pos2d, zero_v, tot_buf, totals_local,
                 presence_sp, totals_sp):
    c = lax.axis_index("c")
    t = lax.axis_index("s")
    iota = lax.iota(jnp.int32, 16)
    zero16 = jnp.zeros((16,), jnp.int32)
    one16 = jnp.ones((16,), jnp.int32)

    pltpu.sync_copy(idx_hbm.at[pl.ds(c * NT + t * TI, TI)], idx_v)

    def _z(i, carry):
        pres_v[pl.ds(i * 16, 16)] = zero16
        return carry
    lax.fori_loop(0, NPAD // 16, _z, 0)
    def _z2(i, carry):
        zero_v[pl.ds(i * 16, 16)] = zero16
        return carry
    lax.fori_loop(0, CH // 16, _z2, 0)

    pltpu.sync_copy(zero_v, ids_hbm.at[pl.ds(c * NPAD + t * CH, CH)])

    def _scat(i, carry):
        iv = idx_v[pl.ds(i * 16, 16)]
        plsc.store_scatter(pres_v, [iv], one16)
        return carry
    lax.fori_loop(0, TI // 16, _scat, 0)

    pltpu.sync_copy(pres_v, presence_sp.at[t])
    plsc.subcore_barrier()

    pltpu.sync_copy(presence_sp.at[:, pl.ds(t * CH, CH)], slice_all)

    g0 = t * CH
    def _merge(j, tot):
        acc = zero16
        for u in range(NS):
            acc = acc | slice_all[u, pl.ds(j * 16, 16)]
        m = (acc > 0) & (g0 + j * 16 + iota < N)
        mi = m.astype(jnp.int32)
        pres_chunk[pl.ds(j * 16, 16)] = mi
        return tot + jnp.sum(mi)
    total_t = lax.fori_loop(0, CH // 16, _merge, jnp.int32(0))

    tot_buf[...] = jnp.broadcast_to(total_t, (16,))
    pltpu.sync_copy(tot_buf, totals_sp.at[pl.ds(t * 16, 16)])
    plsc.subcore_barrier()

    pltpu.sync_copy(totals_sp, totals_local)
    totals_v = plsc.load_gather(totals_local, [iota * 16])
    off0 = jnp.sum(jnp.where(iota < t, totals_v, 0))
    base = c * NPAD + off0
    dump = c * NPAD + N + t

    for k in range(CH // 16):
        lk = k * 16 + iota
        p = jnp.where(lk < total_t, base + lk, dump)
        pos2d[k // 8, pl.ds((k % 8) * 16, 16)] = p

    def _comp(j, off):
        m = pres_chunk[pl.ds(j * 16, 16)] > 0
        ids = g0 + j * 16 + iota
        mi = m.astype(jnp.int32)
        rank = plsc.cumsum(mi)
        posl = jnp.where(m, off + rank - 1, 0)
        plsc.store_scatter(compact_v, [posl], ids, mask=m)
        return off + jnp.sum(mi)
    lax.fori_loop(0, CH // 16, _comp, jnp.int32(0))

    for r in range(CH // 128):
        pltpu.sync_copy(compact_v.at[pl.ds(r * 128, 128)],
                        ids_hbm.at[pos2d.at[r]])


def _gather_body(ids_hbm, scal_hbm, out_hbm, ids_v, scal_all, out_v):
    c = lax.axis_index("c")
    t = lax.axis_index("s")

    pltpu.sync_copy(scal_hbm.at[pl.ds(c * NPAD, NPAD)], scal_all)
    pltpu.sync_copy(ids_hbm.at[pl.ds(c * NPAD + t * CH, CH)], ids_v)

    def _g(j, carry):
        iv = ids_v[pl.ds(j * 16, 16)]
        out_v[pl.ds(j * 16, 16)] = plsc.load_gather(scal_all, [iv])
        return carry
    lax.fori_loop(0, CH // 16, _g, 0)

    pltpu.sync_copy(out_v, out_hbm.at[pl.ds(c * NPAD + t * CH, CH)])


@functools.lru_cache(maxsize=1)
def _sc_kernels():
    mesh = plsc.VectorSubcoreMesh(core_axis_name="c", subcore_axis_name="s",
                                  num_cores=NC, num_subcores=NS)
    params = pltpu.CompilerParams(needs_layout_passes=False)
    unique_k = pl.kernel(
        _unique_body,
        out_type=jax.ShapeDtypeStruct((2 * NPAD,), jnp.int32),
        mesh=mesh,
        compiler_params=params,
        scratch_types=[
            pltpu.VMEM((TI,), jnp.int32),
            pltpu.VMEM((NPAD,), jnp.int32),
            pltpu.VMEM((NS, CH), jnp.int32),
            pltpu.VMEM((CH,), jnp.int32),
            pltpu.VMEM((CH,), jnp.int32),
            pltpu.VMEM((CH // 128, 128), jnp.int32),
            pltpu.VMEM((CH,), jnp.int32),
            pltpu.VMEM((16,), jnp.int32),
            pltpu.VMEM((NS * 16,), jnp.int32),
            pltpu.VMEM_SHARED((NS, NPAD), jnp.int32),
            pltpu.VMEM_SHARED((NS * 16,), jnp.int32),
        ],
    )
    gather_k = pl.kernel(
        _gather_body,
        out_type=jax.ShapeDtypeStruct((2 * NPAD,), jnp.float32),
        mesh=mesh,
        compiler_params=params,
        scratch_types=[
            pltpu.VMEM((CH,), jnp.int32),
            pltpu.VMEM((NPAD,), jnp.float32),
            pltpu.VMEM((CH,), jnp.float32),
        ],
    )
    return unique_k, gather_k


def kernel(adjm, node_feats, trip_od, W1s, W2s, W1d, W2d, Win, bin_, Wout, bout):
    unique_k, gather_k = _sc_kernels()
    idx2 = jnp.concatenate([trip_od[:, 1], trip_od[:, 0]]).astype(jnp.int32)
    ids2 = unique_k(idx2)

    scal_in, scal_out = _embed_call(
        adjm, node_feats, W1s, W2s, W1d, W2d,
        Win, bin_.reshape(1, 1), Wout, bout.reshape(1, 1))
    pad = NPAD - N
    scal2 = jnp.concatenate([
        jnp.pad(scal_in[:, 0], (0, pad)),
        jnp.pad(scal_out[:, 0], (0, pad)),
    ])

    out_flat = gather_k(ids2, scal2)
    return jnp.stack([out_flat[:N], out_flat[NPAD:NPAD + N]], axis=1)


# speedup vs baseline: 4.5792x; 1.0143x over previous
import functools

import jax
import jax.numpy as jnp
from jax import lax
from jax.experimental import pallas as pl
from jax.experimental.pallas import tpu as pltpu
from jax.experimental.pallas import tpu_sc as plsc

N = 10000
NT = 320000
HID = 32
OUT_DIM = 24

NS = 16
NC = 2
CH = 640
NPAD = NS * CH
TI = NT // NS
BM = 400


def _embed_body(adj_ref, nf_ref, w1s_ref, w2s_ref, w1d_ref, w2d_ref,
                win_ref, bin_ref, wout_ref, bout_ref, si_ref, so_ref):
    h = jnp.dot(adj_ref[...], nf_ref[...], preferred_element_type=jnp.float32)
    vs = jnp.dot(w2s_ref[...], wout_ref[...], preferred_element_type=jnp.float32)
    vd = jnp.dot(w2d_ref[...], win_ref[...], preferred_element_type=jnp.float32)
    a_s = jnp.tanh(h * w1s_ref[...])
    a_d = jnp.tanh(h * w1d_ref[...])
    so_ref[...] = jnp.dot(a_s, vs, preferred_element_type=jnp.float32) + bout_ref[0, 0]
    si_ref[...] = jnp.dot(a_d, vd, preferred_element_type=jnp.float32) + bin_ref[0, 0]


def _embed_call(adjm, nf, W1s, W2s, W1d, W2d, Win, bin2, Wout, bout2):
    const = lambda i: (0, 0)
    return pl.pallas_call(
        _embed_body,
        grid=(N // BM,),
        in_specs=[
            pl.BlockSpec((BM, N), lambda i: (i, 0)),
            pl.BlockSpec((N, 1), const),
            pl.BlockSpec((1, HID), const),
            pl.BlockSpec((HID, OUT_DIM), const),
            pl.BlockSpec((1, HID), const),
            pl.BlockSpec((HID, OUT_DIM), const),
            pl.BlockSpec((OUT_DIM, 1), const),
            pl.BlockSpec((1, 1), const),
            pl.BlockSpec((OUT_DIM, 1), const),
            pl.BlockSpec((1, 1), const),
        ],
        out_specs=[pl.BlockSpec((BM, 1), lambda i: (i, 0)),
                   pl.BlockSpec((BM, 1), lambda i: (i, 0))],
        out_shape=[jax.ShapeDtypeStruct((N, 1), jnp.float32),
                   jax.ShapeDtypeStruct((N, 1), jnp.float32)],
    )(adjm, nf, W1s, W2s, W1d, W2d, Win, bin2, Wout, bout2)


def _unique_body(idx_hbm, ids_hbm, idx_v, pres_v, slice_all, pres_chunk,
                 compact_v, pos2d, zero_v, tot_buf, totals_local,
                 presence_sp, totals_sp, sem):
    c = lax.axis_index("c")
    t = lax.axis_index("s")
    iota = lax.iota(jnp.int32, 16)
    zero16 = jnp.zeros((16,), jnp.int32)
    one16 = jnp.ones((16,), jnp.int32)

    idx_cp = pltpu.async_copy(idx_hbm.at[pl.ds(c * NT + t * TI, TI)],
                              idx_v, sem)

    def _z(i, carry):
        for k in range(5):
            pres_v[pl.ds(i * 80 + k * 16, 16)] = zero16
        return carry
    lax.fori_loop(0, NPAD // 80, _z, 0)
    def _z2(i, carry):
        for k in range(5):
            zero_v[pl.ds(i * 80 + k * 16, 16)] = zero16
        return carry
    lax.fori_loop(0, CH // 80, _z2, 0)

    pltpu.sync_copy(zero_v, ids_hbm.at[pl.ds(c * NPAD + t * CH, CH)])
    idx_cp.wait()

    def _scat(i, carry):
        for k in range(5):
            iv = idx_v[pl.ds(i * 80 + k * 16, 16)]
            plsc.store_scatter(pres_v, [iv], one16)
        return carry
    lax.fori_loop(0, TI // 80, _scat, 0)

    pltpu.sync_copy(pres_v, presence_sp.at[t])
    plsc.subcore_barrier()

    pltpu.sync_copy(presence_sp.at[:, pl.ds(t * CH, CH)], slice_all)

    g0 = t * CH
    def _merge(j, tot):
        acc = zero16
        for u in range(NS):
            acc = acc | slice_all[u, pl.ds(j * 16, 16)]
        m = (acc > 0) & (g0 + j * 16 + iota < N)
        mi = m.astype(jnp.int32)
        pres_chunk[pl.ds(j * 16, 16)] = mi
        return tot + jnp.sum(mi)
    total_t = lax.fori_loop(0, CH // 16, _merge, jnp.int32(0))

    tot_buf[...] = jnp.broadcast_to(total_t, (16,))
    pltpu.sync_copy(tot_buf, totals_sp.at[pl.ds(t * 16, 16)])
    plsc.subcore_barrier()

    pltpu.sync_copy(totals_sp, totals_local)
    totals_v = plsc.load_gather(totals_local, [iota * 16])
    off0 = jnp.sum(jnp.where(iota < t, totals_v, 0))
    base = c * NPAD + off0
    dump = c * NPAD + N + t

    for k in range(CH // 16):
        lk = k * 16 + iota
        p = jnp.where(lk < total_t, base + lk, dump)
        pos2d[k // 8, pl.ds((k % 8) * 16, 16)] = p

    def _comp(j, off):
        m = pres_chunk[pl.ds(j * 16, 16)] > 0
        ids = g0 + j * 16 + iota
        mi = m.astype(jnp.int32)
        rank = plsc.cumsum(mi)
        posl = jnp.where(m, off + rank - 1, 0)
        plsc.store_scatter(compact_v, [posl], ids, mask=m)
        return off + jnp.sum(mi)
    lax.fori_loop(0, CH // 16, _comp, jnp.int32(0))

    handles = [pltpu.async_copy(compact_v.at[pl.ds(r * 128, 128)],
                                ids_hbm.at[pos2d.at[r]], sem)
               for r in range(CH // 128)]
    for h in handles:
        h.wait()


def _gather_body(ids_hbm, scal_hbm, out_hbm, ids_v, scal_all, out_v):
    c = lax.axis_index("c")
    t = lax.axis_index("s")

    pltpu.sync_copy(scal_hbm.at[pl.ds(c * NPAD, NPAD)], scal_all)
    pltpu.sync_copy(ids_hbm.at[pl.ds(c * NPAD + t * CH, CH)], ids_v)

    def _g(j, carry):
        iv = ids_v[pl.ds(j * 16, 16)]
        out_v[pl.ds(j * 16, 16)] = plsc.load_gather(scal_all, [iv])
        return carry
    lax.fori_loop(0, CH // 16, _g, 0)

    pltpu.sync_copy(out_v, out_hbm.at[pl.ds(c * NPAD + t * CH, CH)])


@functools.lru_cache(maxsize=1)
def _sc_kernels():
    mesh = plsc.VectorSubcoreMesh(core_axis_name="c", subcore_axis_name="s",
                                  num_cores=NC, num_subcores=NS)
    params = pltpu.CompilerParams(needs_layout_passes=False)
    unique_k = pl.kernel(
        _unique_body,
        out_type=jax.ShapeDtypeStruct((2 * NPAD,), jnp.int32),
        mesh=mesh,
        compiler_params=params,
        scratch_types=[
            pltpu.VMEM((TI,), jnp.int32),
            pltpu.VMEM((NPAD,), jnp.int32),
            pltpu.VMEM((NS, CH), jnp.int32),
            pltpu.VMEM((CH,), jnp.int32),
            pltpu.VMEM((CH,), jnp.int32),
            pltpu.VMEM((CH // 128, 128), jnp.int32),
            pltpu.VMEM((CH,), jnp.int32),
            pltpu.VMEM((16,), jnp.int32),
            pltpu.VMEM((NS * 16,), jnp.int32),
            pltpu.VMEM_SHARED((NS, NPAD), jnp.int32),
            pltpu.VMEM_SHARED((NS * 16,), jnp.int32),
            pltpu.SemaphoreType.DMA,
        ],
    )
    gather_k = pl.kernel(
        _gather_body,
        out_type=jax.ShapeDtypeStruct((2 * NPAD,), jnp.float32),
        mesh=mesh,
        compiler_params=params,
        scratch_types=[
            pltpu.VMEM((CH,), jnp.int32),
            pltpu.VMEM((NPAD,), jnp.float32),
            pltpu.VMEM((CH,), jnp.float32),
        ],
    )
    return unique_k, gather_k


def kernel(adjm, node_feats, trip_od, W1s, W2s, W1d, W2d, Win, bin_, Wout, bout):
    unique_k, gather_k = _sc_kernels()
    idx2 = jnp.concatenate([trip_od[:, 1], trip_od[:, 0]]).astype(jnp.int32)
    ids2 = unique_k(idx2)

    scal_in, scal_out = _embed_call(
        adjm, node_feats, W1s, W2s, W1d, W2d,
        Win, bin_.reshape(1, 1), Wout, bout.reshape(1, 1))
    pad = NPAD - N
    scal2 = jnp.concatenate([
        jnp.pad(scal_in[:, 0], (0, pad)),
        jnp.pad(scal_out[:, 0], (0, pad)),
    ])

    out_flat = gather_k(ids2, scal2)
    return jnp.stack([out_flat[:N], out_flat[NPAD:NPAD + N]], axis=1)
